# baseline (speedup 1.0000x reference)
import functools

import jax
import jax.numpy as jnp
from jax import lax
from jax.experimental import pallas as pl
from jax.experimental.pallas import tpu as pltpu
from jax.experimental.pallas import tpu_sc as plsc

B = 16384
D = 64
N_NEG = 20
L = 16
NC, NS = 2, 16
NW = NC * NS
V = 1000000
BLK = 128
NFULL = V // BLK
K1_ITERS = NFULL // NW + 1

PER_W = B // NW
C = 16
N_CHUNKS = PER_W // C


def _transpose_body(itab_t, tail_pd, ipk, src, dst, sems, osems):
    wid = lax.axis_index("s") * NC + lax.axis_index("c")
    lane = jnp.arange(L, dtype=jnp.int32)

    def fire(k, p):
        b = k * NW + wid

        @pl.when(b < NFULL)
        def _():
            pltpu.async_copy(
                itab_t.at[pl.ds(0, D),
                          pl.ds(pl.multiple_of(b * BLK, BLK), BLK)],
                src[p], sems[p])

    def wait_in(k, p):
        b = k * NW + wid

        @pl.when(b < NFULL)
        def _():
            pltpu.make_async_copy(itab_t.at[pl.ds(0, D), pl.ds(0, BLK)],
                                  src[p], sems[p]).wait()

    def transpose(k, p):
        b = k * NW + wid

        @pl.when(b < NFULL)
        def _():
            def row(lr, carry):
                col = jnp.full((L,), lr, jnp.int32)
                for q in range(D // L):
                    v = plsc.load_gather(src[p], [lane + q * L, col])
                    dst[p][lr, pl.ds(q * L, L)] = v
                return carry

            lax.fori_loop(0, BLK, row, 0, unroll=2)
            pltpu.async_copy(
                dst[p],
                ipk.at[pl.ds(pl.multiple_of(b * BLK, BLK), BLK)],
                osems[p])

    def wait_out(k, p):
        b = k * NW + wid

        @pl.when(b < NFULL)
        def _():
            pltpu.make_async_copy(dst[p], ipk.at[pl.ds(0, BLK)],
                                  osems[p]).wait()

    fire(0, 0)

    def pair(kp, carry):
        k0 = kp * 2
        fire(k0 + 1, 1)
        wait_in(k0, 0)
        transpose(k0, 0)
        fire(k0 + 2, 0)
        wait_in(k0 + 1, 1)
        transpose(k0 + 1, 1)
        wait_out(k0, 0)
        wait_out(k0 + 1, 1)
        return carry

    lax.fori_loop(0, K1_ITERS // 2, pair, 0)
    wait_in(K1_ITERS - 1, 0)
    transpose(K1_ITERS - 1, 0)
    wait_out(K1_ITERS - 1, 0)

    @pl.when(wid == 0)
    def _():
        pltpu.sync_copy(tail_pd, src[0])
        pltpu.sync_copy(src[0], ipk.at[pl.ds(NFULL * BLK, V - NFULL * BLK)])


def _mf_body(pos_hbm, neg_hbm, upk, ipk, out_hbm,
             pidx, nidx, urows, prows, nrows, outv, sems):
    wid = lax.axis_index("s") * NC + lax.axis_index("c")
    base = wid * PER_W
    lane = jnp.arange(L, dtype=jnp.int32)
    NG = C * N_NEG // 80

    def fire(c, p):
        off = base + c * C
        pltpu.sync_copy(pos_hbm.at[pl.ds(off, C)], pidx[p])
        pltpu.sync_copy(neg_hbm.at[pl.ds(off * N_NEG, C * N_NEG)], nidx[p])
        pltpu.async_copy(upk.at[pl.ds(pl.multiple_of(off // 2, C // 2),
                                      C // 2)], urows[p], sems[p])
        pltpu.async_copy(ipk.at[pidx[p]], prows[p], sems[p])
        for k in range(NG):
            pltpu.async_copy(ipk.at[nidx[p].at[pl.ds(k * 80, 80)]],
                             nrows[p].at[pl.ds(k * 80, 80)], sems[p])

    def wait_all(p):
        pltpu.make_async_copy(upk.at[pl.ds(0, C // 2)], urows[p],
                              sems[p]).wait()
        pltpu.make_async_copy(ipk.at[pidx[p]], prows[p], sems[p]).wait()
        for k in range(NG):
            pltpu.make_async_copy(ipk.at[nidx[p].at[pl.ds(k * 80, 80)]],
                                  nrows[p].at[pl.ds(k * 80, 80)],
                                  sems[p]).wait()

    def compute(c, p):
        off = base + c * C
        zero = jnp.zeros((L,), jnp.float32)

        def elem(i, carry):
            ucol = (i & 1) * D
            u = [urows[p][i >> 1, pl.ds(ucol + q * L, L)]
                 for q in range(D // L)]
            pv = [prows[p][i, pl.ds(q * L, L)] for q in range(D // L)]
            pos_sc = jnp.sum(u[0] * pv[0] + u[1] * pv[1]
                             + u[2] * pv[2] + u[3] * pv[3])
            res0 = zero
            res1 = zero
            for j in range(N_NEG):
                r = i * N_NEG + j
                nv = [nrows[p][r, pl.ds(q * L, L)] for q in range(D // L)]
                ns = jnp.sum(u[0] * nv[0] + u[1] * nv[1]
                             + u[2] * nv[2] + u[3] * nv[3])
                r_splat = jnp.full((L,), pos_sc - ns)
                if j < L:
                    res0 = jnp.where(lane == j, r_splat, res0)
                else:
                    res1 = jnp.where(lane == (j - L), r_splat, res1)
            outv[p][pl.ds(i * N_NEG, L)] = res0
            plsc.store_scatter(outv[p],
                               [i * N_NEG + L + (lane & (N_NEG - L - 1))],
                               res1, mask=lane < (N_NEG - L))
            return carry

        lax.fori_loop(0, C, elem, 0)
        pltpu.sync_copy(outv[p], out_hbm.at[pl.ds(off * N_NEG, C * N_NEG)])

    fire(0, 0)

    def pair_body(cp, carry):
        c0 = cp * 2
        fire(c0 + 1, 1)
        wait_all(0)
        compute(c0, 0)

        @pl.when(cp < N_CHUNKS // 2 - 1)
        def _():
            fire(c0 + 2, 0)

        wait_all(1)
        compute(c0 + 1, 1)
        return carry

    lax.fori_loop(0, N_CHUNKS // 2, pair_body, 0)


@jax.jit
def _mf(user, pos_item, neg_flat, user_embed, item_embed):
    mesh = plsc.VectorSubcoreMesh(core_axis_name="c", subcore_axis_name="s",
                                  num_cores=NC, num_subcores=NS)
    cparams = pltpu.CompilerParams(needs_layout_passes=False,
                                   use_tc_tiling_on_sc=True)

    itab_t = item_embed.T
    tail_pd = jnp.pad(item_embed[NFULL * BLK:], ((0, 0), (0, D)))
    u = jnp.take(user_embed, user, axis=0)
    u_pk = u.reshape(B // 2, 2 * D)

    k1 = pl.kernel(
        _transpose_body,
        out_type=jax.ShapeDtypeStruct((V, 2 * D), jnp.float32),
        mesh=mesh,
        compiler_params=cparams,
        scratch_types=[
            [pltpu.VMEM((D, BLK), jnp.float32)] * 2,
            [pltpu.VMEM((BLK, 2 * D), jnp.float32)] * 2,
            [pltpu.SemaphoreType.DMA] * 2,
            [pltpu.SemaphoreType.DMA] * 2,
        ],
    )
    ipk = k1(itab_t, tail_pd)

    k2 = pl.kernel(
        _mf_body,
        out_type=jax.ShapeDtypeStruct((B * N_NEG,), jnp.float32),
        mesh=mesh,
        compiler_params=cparams,
        scratch_types=[
            [pltpu.VMEM((C,), jnp.int32)] * 2,
            [pltpu.VMEM((C * N_NEG,), jnp.int32)] * 2,
            [pltpu.VMEM((C // 2, 2 * D), jnp.float32)] * 2,
            [pltpu.VMEM((C, 2 * D), jnp.float32)] * 2,
            [pltpu.VMEM((C * N_NEG, 2 * D), jnp.float32)] * 2,
            [pltpu.VMEM((C * N_NEG,), jnp.float32)] * 2,
            [pltpu.SemaphoreType.DMA] * 2,
        ],
    )
    out_flat = k2(pos_item, neg_flat, u_pk, ipk)
    return out_flat.reshape(B, N_NEG)


def kernel(user, pos_item, neg_item, user_embed, item_embed):
    user = user.astype(jnp.int32)
    pos_item = pos_item.astype(jnp.int32)
    neg_flat = neg_item.astype(jnp.int32).reshape(B * N_NEG)
    return _mf(user, pos_item, neg_flat, user_embed, item_embed)


# speedup vs baseline: 1.1697x; 1.1697x over previous
import functools

import jax
import jax.numpy as jnp
from jax import lax
from jax.experimental import pallas as pl
from jax.experimental.pallas import tpu as pltpu
from jax.experimental.pallas import tpu_sc as plsc

B = 16384
D = 64
N_NEG = 20
L = 16
NC, NS = 2, 16
NW = NC * NS
V = 1000000
BLK = 128
NFULL = V // BLK
K1_ITERS = NFULL // NW + 1

PER_W = B // NW
C = 16
N_CHUNKS = PER_W // C


def _transpose_body(itab_t, tail_pd, ipk, src, dst, rotv, sems, osems):
    wid = lax.axis_index("s") * NC + lax.axis_index("c")
    lane = jnp.arange(L, dtype=jnp.int32)
    for s in range(L):
        rotv[s, pl.ds(0, L)] = (lane + s) & (L - 1)

    def fire(k, p):
        b = k * NW + wid

        @pl.when(b < NFULL)
        def _():
            pltpu.async_copy(
                itab_t.at[pl.ds(0, D),
                          pl.ds(pl.multiple_of(b * BLK, BLK), BLK)],
                src[p], sems[p])

    def wait_in(k, p):
        b = k * NW + wid

        @pl.when(b < NFULL)
        def _():
            pltpu.make_async_copy(itab_t.at[pl.ds(0, D), pl.ds(0, BLK)],
                                  src[p], sems[p]).wait()

    def transpose(k, p):
        b = k * NW + wid

        @pl.when(b < NFULL)
        def _():
            def cgroup(cg, carry):
                c0 = cg * L
                for dg in range(D // L):
                    dvec = lane + dg * L
                    for s in range(L):
                        cidx = rotv[s, pl.ds(0, L)] + c0
                        v = plsc.load_gather(src[p], [dvec, cidx])
                        plsc.store_scatter(dst[p], [cidx, dvec], v)
                return carry

            lax.fori_loop(0, BLK // L, cgroup, 0)
            pltpu.async_copy(
                dst[p],
                ipk.at[pl.ds(pl.multiple_of(b * BLK, BLK), BLK)],
                osems[p])

    def wait_out(k, p):
        b = k * NW + wid

        @pl.when(b < NFULL)
        def _():
            pltpu.make_async_copy(dst[p], ipk.at[pl.ds(0, BLK)],
                                  osems[p]).wait()

    fire(0, 0)

    def pair(kp, carry):
        k0 = kp * 2
        fire(k0 + 1, 1)
        wait_in(k0, 0)
        transpose(k0, 0)
        fire(k0 + 2, 0)
        wait_in(k0 + 1, 1)
        transpose(k0 + 1, 1)
        wait_out(k0, 0)
        wait_out(k0 + 1, 1)
        return carry

    lax.fori_loop(0, K1_ITERS // 2, pair, 0)
    wait_in(K1_ITERS - 1, 0)
    transpose(K1_ITERS - 1, 0)
    wait_out(K1_ITERS - 1, 0)

    @pl.when(wid == 0)
    def _():
        pltpu.sync_copy(tail_pd, src[0])
        pltpu.sync_copy(src[0], ipk.at[pl.ds(NFULL * BLK, V - NFULL * BLK)])


def _mf_body(pos_hbm, neg_hbm, upk, ipk, out_hbm,
             pidx, nidx, urows, prows, nrows, outv, sems):
    wid = lax.axis_index("s") * NC + lax.axis_index("c")
    base = wid * PER_W
    lane = jnp.arange(L, dtype=jnp.int32)
    NG = C * N_NEG // 80

    def fire(c, p):
        off = base + c * C
        pltpu.sync_copy(pos_hbm.at[pl.ds(off, C)], pidx[p])
        pltpu.sync_copy(neg_hbm.at[pl.ds(off * N_NEG, C * N_NEG)], nidx[p])
        pltpu.async_copy(upk.at[pl.ds(pl.multiple_of(off // 2, C // 2),
                                      C // 2)], urows[p], sems[p])
        pltpu.async_copy(ipk.at[pidx[p]], prows[p], sems[p])
        for k in range(NG):
            pltpu.async_copy(ipk.at[nidx[p].at[pl.ds(k * 80, 80)]],
                             nrows[p].at[pl.ds(k * 80, 80)], sems[p])

    def wait_all(p):
        pltpu.make_async_copy(upk.at[pl.ds(0, C // 2)], urows[p],
                              sems[p]).wait()
        pltpu.make_async_copy(ipk.at[pidx[p]], prows[p], sems[p]).wait()
        for k in range(NG):
            pltpu.make_async_copy(ipk.at[nidx[p].at[pl.ds(k * 80, 80)]],
                                  nrows[p].at[pl.ds(k * 80, 80)],
                                  sems[p]).wait()

    def compute(c, p):
        off = base + c * C
        zero = jnp.zeros((L,), jnp.float32)

        def elem(i, carry):
            ucol = (i & 1) * D
            u = [urows[p][i >> 1, pl.ds(ucol + q * L, L)]
                 for q in range(D // L)]
            pv = [prows[p][i, pl.ds(q * L, L)] for q in range(D // L)]
            pos_sc = jnp.sum(u[0] * pv[0] + u[1] * pv[1]
                             + u[2] * pv[2] + u[3] * pv[3])
            res0 = zero
            res1 = zero
            for j in range(N_NEG):
                r = i * N_NEG + j
                nv = [nrows[p][r, pl.ds(q * L, L)] for q in range(D // L)]
                ns = jnp.sum(u[0] * nv[0] + u[1] * nv[1]
                             + u[2] * nv[2] + u[3] * nv[3])
                r_splat = jnp.full((L,), pos_sc - ns)
                if j < L:
                    res0 = jnp.where(lane == j, r_splat, res0)
                else:
                    res1 = jnp.where(lane == (j - L), r_splat, res1)
            outv[p][pl.ds(i * N_NEG, L)] = res0
            plsc.store_scatter(outv[p],
                               [i * N_NEG + L + (lane & (N_NEG - L - 1))],
                               res1, mask=lane < (N_NEG - L))
            return carry

        lax.fori_loop(0, C, elem, 0)
        pltpu.sync_copy(outv[p], out_hbm.at[pl.ds(off * N_NEG, C * N_NEG)])

    fire(0, 0)

    def pair_body(cp, carry):
        c0 = cp * 2
        fire(c0 + 1, 1)
        wait_all(0)
        compute(c0, 0)

        @pl.when(cp < N_CHUNKS // 2 - 1)
        def _():
            fire(c0 + 2, 0)

        wait_all(1)
        compute(c0 + 1, 1)
        return carry

    lax.fori_loop(0, N_CHUNKS // 2, pair_body, 0)


@jax.jit
def _mf(user, pos_item, neg_flat, user_embed, item_embed):
    mesh = plsc.VectorSubcoreMesh(core_axis_name="c", subcore_axis_name="s",
                                  num_cores=NC, num_subcores=NS)
    cparams = pltpu.CompilerParams(needs_layout_passes=False,
                                   use_tc_tiling_on_sc=True)

    itab_t = item_embed.T
    tail_pd = jnp.pad(item_embed[NFULL * BLK:], ((0, 0), (0, D)))
    u_cols = jnp.take(user_embed.T, user, axis=1)
    u_pk = u_cols.T.reshape(B // 2, 2 * D)

    k1 = pl.kernel(
        _transpose_body,
        out_type=jax.ShapeDtypeStruct((V, 2 * D), jnp.float32),
        mesh=mesh,
        compiler_params=cparams,
        scratch_types=[
            [pltpu.VMEM((D, BLK), jnp.float32)] * 2,
            [pltpu.VMEM((BLK, 2 * D), jnp.float32)] * 2,
            pltpu.VMEM((L, L), jnp.int32),
            [pltpu.SemaphoreType.DMA] * 2,
            [pltpu.SemaphoreType.DMA] * 2,
        ],
    )
    ipk = k1(itab_t, tail_pd)

    k2 = pl.kernel(
        _mf_body,
        out_type=jax.ShapeDtypeStruct((B * N_NEG,), jnp.float32),
        mesh=mesh,
        compiler_params=cparams,
        scratch_types=[
            [pltpu.VMEM((C,), jnp.int32)] * 2,
            [pltpu.VMEM((C * N_NEG,), jnp.int32)] * 2,
            [pltpu.VMEM((C // 2, 2 * D), jnp.float32)] * 2,
            [pltpu.VMEM((C, 2 * D), jnp.float32)] * 2,
            [pltpu.VMEM((C * N_NEG, 2 * D), jnp.float32)] * 2,
            [pltpu.VMEM((C * N_NEG,), jnp.float32)] * 2,
            [pltpu.SemaphoreType.DMA] * 2,
        ],
    )
    out_flat = k2(pos_item, neg_flat, u_pk, ipk)
    return out_flat.reshape(B, N_NEG)


def kernel(user, pos_item, neg_item, user_embed, item_embed):
    user = user.astype(jnp.int32)
    pos_item = pos_item.astype(jnp.int32)
    neg_flat = neg_item.astype(jnp.int32).reshape(B * N_NEG)
    return _mf(user, pos_item, neg_flat, user_embed, item_embed)


# speedup vs baseline: 2.0366x; 1.7411x over previous
import functools

import jax
import jax.numpy as jnp
from jax import lax
from jax.experimental import pallas as pl
from jax.experimental.pallas import tpu as pltpu
from jax.experimental.pallas import tpu_sc as plsc

B = 16384
D = 64
N_NEG = 20
L = 16
NC, NS = 2, 16
NW = NC * NS
PER_W = B // NW
C = 32
N_CHUNKS = PER_W // C
NEG_IW = 128
NEG_ROWS_C = C * N_NEG // NEG_IW


def _mf_body(pos_hbm, neg_hbm, upk, itab, out_hbm,
             pidx, nidx, urows, prows, nrows, outv, sems):
    wid = lax.axis_index("s") * NC + lax.axis_index("c")
    base = wid * PER_W

    def fire(c, p):
        off = base + c * C
        pltpu.sync_copy(pos_hbm.at[pl.ds(off, C)], pidx[p])
        for k in range(NEG_ROWS_C):
            pltpu.sync_copy(
                neg_hbm.at[pl.ds(off * N_NEG + k * NEG_IW, NEG_IW)],
                nidx[p].at[k])
        pltpu.async_copy(upk.at[pl.ds(off // 2, C // 2)], urows[p], sems[p])
        pltpu.async_copy(itab.at[pidx[p]], prows[p], sems[p])
        for k in range(NEG_ROWS_C):
            pltpu.async_copy(itab.at[nidx[p].at[k]],
                             nrows[p].at[pl.ds(k * NEG_IW, NEG_IW)],
                             sems[p])

    def wait_all(p):
        pltpu.make_async_copy(upk.at[pl.ds(0, C // 2)], urows[p],
                              sems[p]).wait()
        pltpu.make_async_copy(itab.at[pidx[p]], prows[p], sems[p]).wait()
        for k in range(NEG_ROWS_C):
            pltpu.make_async_copy(itab.at[nidx[p].at[k]],
                                  nrows[p].at[pl.ds(k * NEG_IW, NEG_IW)],
                                  sems[p]).wait()

    def compute(c, p):
        off = base + c * C
        lane = jnp.arange(L, dtype=jnp.int32)
        zero = jnp.zeros((L,), jnp.float32)

        def elem(i, carry):
            ucol = (i & 1) * D
            u = [urows[p][i >> 1, pl.ds(ucol + q * L, L)]
                 for q in range(D // L)]
            pv = [prows[p][i, pl.ds(q * L, L)] for q in range(D // L)]
            pos_sc = jnp.sum(u[0] * pv[0] + u[1] * pv[1]
                             + u[2] * pv[2] + u[3] * pv[3])
            res0 = zero
            res1 = zero
            for j in range(N_NEG):
                r = i * N_NEG + j
                nv = [nrows[p][r, pl.ds(q * L, L)] for q in range(D // L)]
                ns = jnp.sum(u[0] * nv[0] + u[1] * nv[1]
                             + u[2] * nv[2] + u[3] * nv[3])
                r_splat = jnp.full((L,), pos_sc - ns)
                if j < L:
                    res0 = jnp.where(lane == j, r_splat, res0)
                else:
                    res1 = jnp.where(lane == (j - L), r_splat, res1)
            outv[p][i, pl.ds(0, L)] = res0
            plsc.store_scatter(outv[p],
                               [jnp.full((L,), i, jnp.int32),
                                L + (lane & (N_NEG - L - 1))],
                               res1, mask=lane < (N_NEG - L))
            return carry

        lax.fori_loop(0, C, elem, 0)
        pltpu.sync_copy(outv[p], out_hbm.at[pl.ds(off, C)])

    fire(0, 0)

    def pair_body(cp, carry):
        c0 = cp * 2
        fire(c0 + 1, 1)
        wait_all(0)
        compute(c0, 0)

        @pl.when(cp < N_CHUNKS // 2 - 1)
        def _():
            fire(c0 + 2, 0)

        wait_all(1)
        compute(c0 + 1, 1)
        return carry

    lax.fori_loop(0, N_CHUNKS // 2, pair_body, 0)


@jax.jit
def _mf(user, pos_item, neg_flat, user_embed, item_embed):
    mesh = plsc.VectorSubcoreMesh(core_axis_name="c", subcore_axis_name="s",
                                  num_cores=NC, num_subcores=NS)
    u = jnp.take(user_embed, user, axis=0)
    u_pk = u.reshape(B // 2, 2 * D)
    run = pl.kernel(
        _mf_body,
        out_type=jax.ShapeDtypeStruct((B, N_NEG), jnp.float32),
        mesh=mesh,
        compiler_params=pltpu.CompilerParams(needs_layout_passes=False,
                                             use_tc_tiling_on_sc=False),
        scratch_types=[
            [pltpu.VMEM((C,), jnp.int32)] * 2,
            [pltpu.VMEM((NEG_ROWS_C, NEG_IW), jnp.int32)] * 2,
            [pltpu.VMEM((C // 2, 2 * D), jnp.float32)] * 2,
            [pltpu.VMEM((C, D), jnp.float32)] * 2,
            [pltpu.VMEM((C * N_NEG, D), jnp.float32)] * 2,
            [pltpu.VMEM((C, N_NEG), jnp.float32)] * 2,
            [pltpu.SemaphoreType.DMA] * 2,
        ],
    )
    return run(pos_item, neg_flat, u_pk, item_embed)


def kernel(user, pos_item, neg_item, user_embed, item_embed):
    user = user.astype(jnp.int32)
    pos_item = pos_item.astype(jnp.int32)
    neg_flat = neg_item.astype(jnp.int32).reshape(B * N_NEG)
    return _mf(user, pos_item, neg_flat, user_embed, item_embed)
